# baseline (speedup 1.0000x reference)
import jax
import jax.numpy as jnp
from jax import lax
from jax.experimental import pallas as pl
from jax.experimental.pallas import tpu as pltpu
from jax.experimental.pallas import tpu_sc as plsc

BATCH = 16384
D = 64
NC = 2
NS = 16
NW = NC * NS
BPW = BATCH // NW
S = 128
NSUB = BPW // S


def _body(uid_h, ug_h, ua_h, uo_h, idt_h, gt_h, at_h, ot_h,
          o0_h, o1_h, o2_h, o3_h,
          ids_v, idx1, idx2, idx3, bid, b1, b2, b3, sem, sem2):
    wid = lax.axis_index("s") * NC + lax.axis_index("c")
    base = wid * BPW
    base_w = wid * NSUB
    pltpu.sync_copy(uid_h.at[pl.ds(base, BPW)], ids_v)
    pltpu.sync_copy(ug_h.at[pl.ds(base_w, NSUB)], idx1)
    pltpu.sync_copy(ua_h.at[pl.ds(base_w, NSUB)], idx2)
    pltpu.sync_copy(uo_h.at[pl.ds(base_w, NSUB)], idx3)

    def fire(g, _):
        v = ids_v[pl.ds(g * 16, 16)]
        for j in range(16):
            u = v[j]
            pltpu.async_copy(idt_h.at[pl.ds(u, 1)],
                             bid.at[pl.ds(g * 16 + j, 1)], sem2)
        return 0

    lax.fori_loop(0, BPW // 16, fire, 0)

    for s in range(NSUB):
        c1 = pltpu.async_copy(gt_h.at[idx1.at[s]], b1, sem)
        c2 = pltpu.async_copy(at_h.at[idx2.at[s]], b2, sem)
        c3 = pltpu.async_copy(ot_h.at[idx3.at[s]], b3, sem)
        c1.wait()
        c2.wait()
        c3.wait()
        pltpu.sync_copy(b1, o1_h.at[pl.ds(base + s * S, S)])
        pltpu.sync_copy(b2, o2_h.at[pl.ds(base + s * S, S)])
        pltpu.sync_copy(b3, o3_h.at[pl.ds(base + s * S, S)])

    def drain(i, _):
        pltpu.make_async_copy(idt_h.at[pl.ds(0, 1)], bid.at[pl.ds(i, 1)], sem2).wait()
        return 0

    lax.fori_loop(0, BPW, drain, 0)
    pltpu.sync_copy(bid, o0_h.at[pl.ds(base, BPW)])


def kernel(user_ids, user_genders, user_ages, user_occs,
           id_table, gender_table, age_table, occ_table):
    mesh = plsc.VectorSubcoreMesh(core_axis_name="c", subcore_axis_name="s")
    k = pl.kernel(
        _body,
        mesh=mesh,
        out_type=(
            jax.ShapeDtypeStruct((BATCH, D), jnp.float32),
            jax.ShapeDtypeStruct((BATCH, 2 * D), jnp.float32),
            jax.ShapeDtypeStruct((BATCH, 2 * D), jnp.float32),
            jax.ShapeDtypeStruct((BATCH, 2 * D), jnp.float32),
        ),
        scratch_types=[
            pltpu.VMEM((BPW,), jnp.int32),
            pltpu.VMEM((NSUB, S), jnp.int32),
            pltpu.VMEM((NSUB, S), jnp.int32),
            pltpu.VMEM((NSUB, S), jnp.int32),
            pltpu.VMEM((BPW, D), jnp.float32),
            pltpu.VMEM((S, 2 * D), jnp.float32),
            pltpu.VMEM((S, 2 * D), jnp.float32),
            pltpu.VMEM((S, 2 * D), jnp.float32),
            pltpu.SemaphoreType.DMA,
            pltpu.SemaphoreType.DMA,
        ],
    )
    r = (BATCH // S, S)
    pad = ((0, 0), (0, D))
    o0, o1, o2, o3 = k(user_ids.astype(jnp.int32),
                       user_genders.astype(jnp.int32).reshape(r),
                       user_ages.astype(jnp.int32).reshape(r),
                       user_occs.astype(jnp.int32).reshape(r),
                       id_table,
                       jnp.pad(gender_table, pad),
                       jnp.pad(age_table, pad),
                       jnp.pad(occ_table, pad))
    return jnp.concatenate(
        [o0, o1[:, :D], o2[:, :D], o3[:, :D]], axis=1)


# speedup vs baseline: 1.0018x; 1.0018x over previous
import jax
import jax.numpy as jnp
from jax import lax
from jax.experimental import pallas as pl
from jax.experimental.pallas import tpu as pltpu
from jax.experimental.pallas import tpu_sc as plsc

BATCH = 16384
D = 64
NC = 2
NS = 16
NW = NC * NS
BPW = BATCH // NW
S = 128
NSUB = BPW // S


def _body(uid_h, ug_h, ua_h, uo_h, idt_h, gt_h, at_h, ot_h,
          o0_h, o1_h, o2_h, o3_h,
          ids_v, idx1, idx2, idx3, bid, b1, b2, b3, sem, sem2):
    wid = lax.axis_index("s") * NC + lax.axis_index("c")
    base = wid * BPW
    base_w = wid * NSUB
    pltpu.sync_copy(uid_h.at[pl.ds(base, BPW)], ids_v)
    pltpu.sync_copy(ug_h.at[pl.ds(base_w, NSUB)], idx1)
    pltpu.sync_copy(ua_h.at[pl.ds(base_w, NSUB)], idx2)
    pltpu.sync_copy(uo_h.at[pl.ds(base_w, NSUB)], idx3)

    def fire(g, _):
        v = ids_v[pl.ds(g * 16, 16)]
        for j in range(16):
            u = v[j]
            pltpu.async_copy(idt_h.at[pl.ds(u, 1)],
                             bid.at[pl.ds(g * 16 + j, 1)], sem2)
        return 0

    lax.fori_loop(0, BPW // 16, fire, 0)

    for s in range(NSUB):
        c1 = pltpu.async_copy(gt_h.at[idx1.at[s]], b1, sem)
        c2 = pltpu.async_copy(at_h.at[idx2.at[s]], b2, sem)
        c3 = pltpu.async_copy(ot_h.at[idx3.at[s]], b3, sem)
        c1.wait()
        c2.wait()
        c3.wait()
        pltpu.sync_copy(b1, o1_h.at[pl.ds(base + s * S, S)])
        pltpu.sync_copy(b2, o2_h.at[pl.ds(base + s * S, S)])
        pltpu.sync_copy(b3, o3_h.at[pl.ds(base + s * S, S)])

    pltpu.make_async_copy(idt_h.at[pl.ds(0, BPW)], bid, sem2).wait()
    pltpu.sync_copy(bid, o0_h.at[pl.ds(base, BPW)])


def kernel(user_ids, user_genders, user_ages, user_occs,
           id_table, gender_table, age_table, occ_table):
    mesh = plsc.VectorSubcoreMesh(core_axis_name="c", subcore_axis_name="s")
    k = pl.kernel(
        _body,
        mesh=mesh,
        out_type=(
            jax.ShapeDtypeStruct((BATCH, D), jnp.float32),
            jax.ShapeDtypeStruct((BATCH, 2 * D), jnp.float32),
            jax.ShapeDtypeStruct((BATCH, 2 * D), jnp.float32),
            jax.ShapeDtypeStruct((BATCH, 2 * D), jnp.float32),
        ),
        scratch_types=[
            pltpu.VMEM((BPW,), jnp.int32),
            pltpu.VMEM((NSUB, S), jnp.int32),
            pltpu.VMEM((NSUB, S), jnp.int32),
            pltpu.VMEM((NSUB, S), jnp.int32),
            pltpu.VMEM((BPW, D), jnp.float32),
            pltpu.VMEM((S, 2 * D), jnp.float32),
            pltpu.VMEM((S, 2 * D), jnp.float32),
            pltpu.VMEM((S, 2 * D), jnp.float32),
            pltpu.SemaphoreType.DMA,
            pltpu.SemaphoreType.DMA,
        ],
    )
    r = (BATCH // S, S)
    pad = ((0, 0), (0, D))
    o0, o1, o2, o3 = k(user_ids.astype(jnp.int32),
                       user_genders.astype(jnp.int32).reshape(r),
                       user_ages.astype(jnp.int32).reshape(r),
                       user_occs.astype(jnp.int32).reshape(r),
                       id_table,
                       jnp.pad(gender_table, pad),
                       jnp.pad(age_table, pad),
                       jnp.pad(occ_table, pad))
    return jnp.concatenate(
        [o0, o1[:, :D], o2[:, :D], o3[:, :D]], axis=1)


# speedup vs baseline: 1.7187x; 1.7156x over previous
import jax
import jax.numpy as jnp
from jax import lax
from jax.experimental import pallas as pl
from jax.experimental.pallas import tpu as pltpu
from jax.experimental.pallas import tpu_sc as plsc

BATCH = 16384
D = 64
NC = 2
NS = 16
NW = NC * NS
BPW = BATCH // NW
S = 128
NSUB = BPW // S


def _body(uid_h, ug_h, ua_h, uo_h, idt_h, gt_h, at_h, ot_h,
          o0_h, o1_h, o2_h, o3_h,
          ids_v, idx1, idx2, idx3, bid, b1, b2, b3, sem, sem2):
    wid = lax.axis_index("s") * NC + lax.axis_index("c")
    base = wid * BPW
    base_w = wid * NSUB
    pltpu.sync_copy(uid_h.at[pl.ds(base, BPW)], ids_v)
    pltpu.sync_copy(ug_h.at[pl.ds(base_w, NSUB)], idx1)
    pltpu.sync_copy(ua_h.at[pl.ds(base_w, NSUB)], idx2)
    pltpu.sync_copy(uo_h.at[pl.ds(base_w, NSUB)], idx3)

    def fire(g, _):
        v = ids_v[pl.ds(g * 16, 16)]
        for j in range(16):
            u = v[j]
            pltpu.async_copy(idt_h.at[pl.ds(u, 1)],
                             bid.at[pl.ds(g * 16 + j, 1)], sem2)
        return 0

    lax.fori_loop(0, BPW // 16, fire, 0)

    for s in range(NSUB):
        c1 = pltpu.async_copy(gt_h.at[idx1.at[s]], b1, sem)
        c2 = pltpu.async_copy(at_h.at[idx2.at[s]], b2, sem)
        c3 = pltpu.async_copy(ot_h.at[idx3.at[s]], b3, sem)
        c1.wait()
        c2.wait()
        c3.wait()
        pltpu.sync_copy(b1, o1_h.at[pl.ds(base + s * S, S)])
        pltpu.sync_copy(b2, o2_h.at[pl.ds(base + s * S, S)])
        pltpu.sync_copy(b3, o3_h.at[pl.ds(base + s * S, S)])

    pltpu.make_async_copy(idt_h.at[pl.ds(0, BPW)], bid, sem2).wait()
    pltpu.sync_copy(bid, o0_h.at[pl.ds(base, BPW)])


def kernel(user_ids, user_genders, user_ages, user_occs,
           id_table, gender_table, age_table, occ_table):
    mesh = plsc.VectorSubcoreMesh(core_axis_name="c", subcore_axis_name="s")
    k = pl.kernel(
        _body,
        mesh=mesh,
        out_type=(
            jax.ShapeDtypeStruct((BATCH, D), jnp.float32),
            jax.ShapeDtypeStruct((BATCH, 2 * D), jnp.float32),
            jax.ShapeDtypeStruct((BATCH, 2 * D), jnp.float32),
            jax.ShapeDtypeStruct((BATCH, 2 * D), jnp.float32),
        ),
        scratch_types=[
            pltpu.VMEM((BPW,), jnp.int32),
            pltpu.VMEM((NSUB, S), jnp.int32),
            pltpu.VMEM((NSUB, S), jnp.int32),
            pltpu.VMEM((NSUB, S), jnp.int32),
            pltpu.VMEM((BPW, D), jnp.float32),
            pltpu.VMEM((S, 2 * D), jnp.float32),
            pltpu.VMEM((S, 2 * D), jnp.float32),
            pltpu.VMEM((S, 2 * D), jnp.float32),
            pltpu.SemaphoreType.DMA,
            pltpu.SemaphoreType.DMA,
        ],
    )
    r = (BATCH // S, S)
    pad = ((0, 0), (0, D))

    def rep(tab):
        return jnp.tile(jnp.pad(tab, pad), (NW, 1))

    def off(idx, nrows):
        w = (jnp.arange(NW, dtype=jnp.int32) * nrows)[:, None]
        return (idx.astype(jnp.int32).reshape(NW, BPW) + w).reshape(r)

    o0, o1, o2, o3 = k(user_ids.astype(jnp.int32),
                       off(user_genders, 2),
                       off(user_ages, 7),
                       off(user_occs, 21),
                       id_table,
                       rep(gender_table),
                       rep(age_table),
                       rep(occ_table))
    return jnp.concatenate(
        [o0, o1[:, :D], o2[:, :D], o3[:, :D]], axis=1)


# speedup vs baseline: 2.4897x; 1.4486x over previous
import jax
import jax.numpy as jnp
from jax import lax
from jax.experimental import pallas as pl
from jax.experimental.pallas import tpu as pltpu
from jax.experimental.pallas import tpu_sc as plsc

BATCH = 16384
D = 64
V = 1000000
NC = 2
NS = 16
NW = NC * NS
BPW = BATCH // NW
S = 128
NSUB = BPW // S
PPC = D // NC
BT = BATCH // NS
GT = BT // S


def _body(uid_h, ug_h, ua_h, uo_h, xT_h, gt_h, at_h, ot_h,
          o0T_h, o1_h, o2_h, o3_h,
          uidx, idx1, idx2, idx3, pbuf, b1, b2, b3, spA,
          sem, sem2, sem3):
    cid = lax.axis_index("c")
    sid = lax.axis_index("s")
    wid = sid * NC + cid
    base = wid * BPW
    base_w = wid * NSUB
    pltpu.sync_copy(uid_h.at[pl.ds(sid * GT, GT)], uidx)
    pltpu.sync_copy(ug_h.at[pl.ds(base_w, NSUB)], idx1)
    pltpu.sync_copy(ua_h.at[pl.ds(base_w, NSUB)], idx2)
    pltpu.sync_copy(uo_h.at[pl.ds(base_w, NSUB)], idx3)

    for s in range(NSUB):
        c1 = pltpu.async_copy(gt_h.at[idx1.at[s]], b1, sem)
        c2 = pltpu.async_copy(at_h.at[idx2.at[s]], b2, sem)
        c3 = pltpu.async_copy(ot_h.at[idx3.at[s]], b3, sem)
        c1.wait()
        c2.wait()
        c3.wait()
        pltpu.sync_copy(b1, o1_h.at[pl.ds(base + s * S, S)])
        pltpu.sync_copy(b2, o2_h.at[pl.ds(base + s * S, S)])
        pltpu.sync_copy(b3, o3_h.at[pl.ds(base + s * S, S)])

    d0 = cid * PPC
    for p in range(PPC):
        @pl.when(sid == 0)
        def _stage():
            pltpu.sync_copy(xT_h.at[d0 + p], spA)

        plsc.subcore_barrier()
        waits = []
        for g in range(GT):
            waits.append(pltpu.async_copy(
                spA.at[uidx.at[g]], pbuf.at[pl.ds(g * S, S)], sem3))
        for w in waits:
            w.wait()
        pltpu.sync_copy(pbuf, o0T_h.at[d0 + p, pl.ds(sid * BT, BT)])
        plsc.subcore_barrier()


def kernel(user_ids, user_genders, user_ages, user_occs,
           id_table, gender_table, age_table, occ_table):
    mesh = plsc.VectorSubcoreMesh(core_axis_name="c", subcore_axis_name="s")
    k = pl.kernel(
        _body,
        mesh=mesh,
        out_type=(
            jax.ShapeDtypeStruct((D, BATCH), jnp.float32),
            jax.ShapeDtypeStruct((BATCH, 2 * D), jnp.float32),
            jax.ShapeDtypeStruct((BATCH, 2 * D), jnp.float32),
            jax.ShapeDtypeStruct((BATCH, 2 * D), jnp.float32),
        ),
        scratch_types=[
            pltpu.VMEM((GT, S), jnp.int32),
            pltpu.VMEM((NSUB, S), jnp.int32),
            pltpu.VMEM((NSUB, S), jnp.int32),
            pltpu.VMEM((NSUB, S), jnp.int32),
            pltpu.VMEM((BT,), jnp.float32),
            pltpu.VMEM((S, 2 * D), jnp.float32),
            pltpu.VMEM((S, 2 * D), jnp.float32),
            pltpu.VMEM((S, 2 * D), jnp.float32),
            pltpu.VMEM_SHARED((V,), jnp.float32),
            pltpu.SemaphoreType.DMA,
            pltpu.SemaphoreType.DMA,
            pltpu.SemaphoreType.DMA,
        ],
    )
    r = (BATCH // S, S)
    pad = ((0, 0), (0, D))

    def rep(tab):
        return jnp.tile(jnp.pad(tab, pad), (NW, 1))

    def off(idx, nrows):
        w = (jnp.arange(NW, dtype=jnp.int32) * nrows)[:, None]
        return (idx.astype(jnp.int32).reshape(NW, BPW) + w).reshape(r)

    o0T, o1, o2, o3 = k(user_ids.astype(jnp.int32).reshape(r),
                        off(user_genders, 2),
                        off(user_ages, 7),
                        off(user_occs, 21),
                        id_table.T,
                        rep(gender_table),
                        rep(age_table),
                        rep(occ_table))
    return jnp.concatenate(
        [o0T.T, o1[:, :D], o2[:, :D], o3[:, :D]], axis=1)


# speedup vs baseline: 2.4981x; 1.0034x over previous
import jax
import jax.numpy as jnp
from jax import lax
from jax.experimental import pallas as pl
from jax.experimental.pallas import tpu as pltpu
from jax.experimental.pallas import tpu_sc as plsc

BATCH = 16384
D = 64
V = 1000000
NC = 2
NS = 16
NW = NC * NS
BPW = BATCH // NW
S = 128
NSUB = BPW // S
PPC = D // NC
BT = BATCH // NS
GT = BT // S


def _body(uid1_h, ug_h, ua_h, uo_h, xT_h, gt_h, at_h, ot_h,
          o0T_h, o1_h, o2_h, o3_h,
          uidx1, idx1, idx2, idx3, pbuf, b1, b2, b3, spA,
          sem, sem2, sem3):
    cid = lax.axis_index("c")
    sid = lax.axis_index("s")
    wid = sid * NC + cid
    base = wid * BPW
    base_w = wid * NSUB
    pltpu.sync_copy(uid1_h.at[pl.ds(sid * BT, BT)], uidx1)
    pltpu.sync_copy(ug_h.at[pl.ds(base_w, NSUB)], idx1)
    pltpu.sync_copy(ua_h.at[pl.ds(base_w, NSUB)], idx2)
    pltpu.sync_copy(uo_h.at[pl.ds(base_w, NSUB)], idx3)

    for s in range(NSUB):
        c1 = pltpu.async_copy(gt_h.at[idx1.at[s]], b1, sem)
        c2 = pltpu.async_copy(at_h.at[idx2.at[s]], b2, sem)
        c3 = pltpu.async_copy(ot_h.at[idx3.at[s]], b3, sem)
        c1.wait()
        c2.wait()
        c3.wait()
        pltpu.sync_copy(b1, o1_h.at[pl.ds(base + s * S, S)])
        pltpu.sync_copy(b2, o2_h.at[pl.ds(base + s * S, S)])
        pltpu.sync_copy(b3, o3_h.at[pl.ds(base + s * S, S)])

    d0 = cid * PPC
    for p in range(PPC):
        @pl.when(sid == 0)
        def _stage():
            pltpu.sync_copy(xT_h.at[d0 + p], spA)

        plsc.subcore_barrier()
        pltpu.async_copy(spA.at[uidx1], pbuf, sem3).wait()
        pltpu.sync_copy(pbuf, o0T_h.at[d0 + p, pl.ds(sid * BT, BT)])
        plsc.subcore_barrier()


def kernel(user_ids, user_genders, user_ages, user_occs,
           id_table, gender_table, age_table, occ_table):
    mesh = plsc.VectorSubcoreMesh(core_axis_name="c", subcore_axis_name="s")
    k = pl.kernel(
        _body,
        mesh=mesh,
        out_type=(
            jax.ShapeDtypeStruct((D, BATCH), jnp.float32),
            jax.ShapeDtypeStruct((BATCH, 2 * D), jnp.float32),
            jax.ShapeDtypeStruct((BATCH, 2 * D), jnp.float32),
            jax.ShapeDtypeStruct((BATCH, 2 * D), jnp.float32),
        ),
        scratch_types=[
            pltpu.VMEM((BT,), jnp.int32),
            pltpu.VMEM((NSUB, S), jnp.int32),
            pltpu.VMEM((NSUB, S), jnp.int32),
            pltpu.VMEM((NSUB, S), jnp.int32),
            pltpu.VMEM((BT,), jnp.float32),
            pltpu.VMEM((S, 2 * D), jnp.float32),
            pltpu.VMEM((S, 2 * D), jnp.float32),
            pltpu.VMEM((S, 2 * D), jnp.float32),
            pltpu.VMEM_SHARED((V,), jnp.float32),
            pltpu.SemaphoreType.DMA,
            pltpu.SemaphoreType.DMA,
            pltpu.SemaphoreType.DMA,
        ],
    )
    r = (BATCH // S, S)
    pad = ((0, 0), (0, D))

    def rep(tab):
        return jnp.tile(jnp.pad(tab, pad), (NW, 1))

    def off(idx, nrows):
        w = (jnp.arange(NW, dtype=jnp.int32) * nrows)[:, None]
        return (idx.astype(jnp.int32).reshape(NW, BPW) + w).reshape(r)

    o0T, o1, o2, o3 = k(user_ids.astype(jnp.int32),
                        off(user_genders, 2),
                        off(user_ages, 7),
                        off(user_occs, 21),
                        id_table.T,
                        rep(gender_table),
                        rep(age_table),
                        rep(occ_table))
    return jnp.concatenate(
        [o0T.T, o1[:, :D], o2[:, :D], o3[:, :D]], axis=1)


# speedup vs baseline: 2.6328x; 1.0539x over previous
import jax
import jax.numpy as jnp
from jax import lax
from jax.experimental import pallas as pl
from jax.experimental.pallas import tpu as pltpu
from jax.experimental.pallas import tpu_sc as plsc

BATCH = 16384
D = 64
V = 1000000
NC = 2
NS = 16
NW = NC * NS
BPW = BATCH // NW
S = 128
NSUB = BPW // S
PPC = D // NC
BT = BATCH // NS
GT = BT // S
CH = 62464
CHT = V - NS * CH


def _body(uid1_h, ug_h, ua_h, uo_h, xT_h, gt_h, at_h, ot_h,
          o0T_h, o1_h, o2_h, o3_h,
          uidx1, idx1, idx2, idx3, pbuf, pbuf2, b1, b2, b3, spA,
          sem, sem2, sem3, semw):
    cid = lax.axis_index("c")
    sid = lax.axis_index("s")
    wid = sid * NC + cid
    base = wid * BPW
    base_w = wid * NSUB
    pltpu.sync_copy(uid1_h.at[pl.ds(sid * BT, BT)], uidx1)
    pltpu.sync_copy(ug_h.at[pl.ds(base_w, NSUB)], idx1)
    pltpu.sync_copy(ua_h.at[pl.ds(base_w, NSUB)], idx2)
    pltpu.sync_copy(uo_h.at[pl.ds(base_w, NSUB)], idx3)

    d0 = cid * PPC
    for p in range(PPC):
        @pl.when(sid == 0)
        def _kick():
            pltpu.async_copy(xT_h.at[d0 + p], spA, sem2)

        if p < NSUB:
            s = p
            c1 = pltpu.async_copy(gt_h.at[idx1.at[s]], b1, sem)
            c2 = pltpu.async_copy(at_h.at[idx2.at[s]], b2, sem)
            c3 = pltpu.async_copy(ot_h.at[idx3.at[s]], b3, sem)
            c1.wait()
            c2.wait()
            c3.wait()
            pltpu.sync_copy(b1, o1_h.at[pl.ds(base + s * S, S)])
            pltpu.sync_copy(b2, o2_h.at[pl.ds(base + s * S, S)])
            pltpu.sync_copy(b3, o3_h.at[pl.ds(base + s * S, S)])

        pb = pbuf if p % 2 == 0 else pbuf2
        if p >= 2:
            pltpu.make_async_copy(
                pb, o0T_h.at[d0 + p - 2, pl.ds(sid * BT, BT)], semw).wait()

        @pl.when(sid == 0)
        def _drain():
            pltpu.make_async_copy(xT_h.at[d0 + p], spA, sem2).wait()

        plsc.subcore_barrier()
        pltpu.async_copy(spA.at[uidx1], pb, sem3).wait()
        plsc.subcore_barrier()
        pltpu.async_copy(pb, o0T_h.at[d0 + p, pl.ds(sid * BT, BT)], semw)

    for p in (PPC - 2, PPC - 1):
        pb = pbuf if p % 2 == 0 else pbuf2
        pltpu.make_async_copy(
            pb, o0T_h.at[d0 + p, pl.ds(sid * BT, BT)], semw).wait()


def kernel(user_ids, user_genders, user_ages, user_occs,
           id_table, gender_table, age_table, occ_table):
    mesh = plsc.VectorSubcoreMesh(core_axis_name="c", subcore_axis_name="s")
    k = pl.kernel(
        _body,
        mesh=mesh,
        out_type=(
            jax.ShapeDtypeStruct((D, BATCH), jnp.float32),
            jax.ShapeDtypeStruct((BATCH, 2 * D), jnp.float32),
            jax.ShapeDtypeStruct((BATCH, 2 * D), jnp.float32),
            jax.ShapeDtypeStruct((BATCH, 2 * D), jnp.float32),
        ),
        scratch_types=[
            pltpu.VMEM((BT,), jnp.int32),
            pltpu.VMEM((NSUB, S), jnp.int32),
            pltpu.VMEM((NSUB, S), jnp.int32),
            pltpu.VMEM((NSUB, S), jnp.int32),
            pltpu.VMEM((BT,), jnp.float32),
            pltpu.VMEM((BT,), jnp.float32),
            pltpu.VMEM((S, 2 * D), jnp.float32),
            pltpu.VMEM((S, 2 * D), jnp.float32),
            pltpu.VMEM((S, 2 * D), jnp.float32),
            pltpu.VMEM_SHARED((V,), jnp.float32),
            pltpu.SemaphoreType.DMA,
            pltpu.SemaphoreType.DMA,
            pltpu.SemaphoreType.DMA,
            pltpu.SemaphoreType.DMA,
        ],
    )
    r = (BATCH // S, S)
    pad = ((0, 0), (0, D))

    def rep(tab):
        return jnp.tile(jnp.pad(tab, pad), (NW, 1))

    def off(idx, nrows):
        w = (jnp.arange(NW, dtype=jnp.int32) * nrows)[:, None]
        return (idx.astype(jnp.int32).reshape(NW, BPW) + w).reshape(r)

    o0T, o1, o2, o3 = k(user_ids.astype(jnp.int32),
                        off(user_genders, 2),
                        off(user_ages, 7),
                        off(user_occs, 21),
                        id_table.T,
                        rep(gender_table),
                        rep(age_table),
                        rep(occ_table))
    return jnp.concatenate(
        [o0T.T, o1[:, :D], o2[:, :D], o3[:, :D]], axis=1)
